# baseline (speedup 1.0000x reference)
import functools

import jax
import jax.numpy as jnp
from jax import lax
from jax.experimental import pallas as pl
from jax.experimental.pallas import tpu as pltpu
from jax.experimental.pallas import tpu_sc as plsc


_BATCH = 8192
_CH = 4096
_LANES = 16
_NW = 32
_ROWS_PW = _BATCH // _NW
_R = 4
_NCHUNK = _ROWS_PW // _R
_NV = _CH // _LANES
_U = 4


def _gather_chunk(perm_v, inb, outb):
    def v_body(g, _):
        for u in range(_U):
            off = g * (_U * _LANES) + u * _LANES
            idx = perm_v[pl.ds(off, _LANES)]
            for r in range(_R):
                rv = jnp.full((_LANES,), r, jnp.int32)
                outb[r, pl.ds(off, _LANES)] = plsc.load_gather(inb, [rv, idx])
        return _

    lax.fori_loop(0, _NV // _U, v_body, None)


def _sc_permute(x, perm32):
    mesh = plsc.VectorSubcoreMesh(core_axis_name="c", subcore_axis_name="s")

    @functools.partial(
        pl.kernel,
        mesh=mesh,
        compiler_params=pltpu.CompilerParams(needs_layout_passes=False),
        out_type=jax.ShapeDtypeStruct((_BATCH, _CH), jnp.float32),
        scratch_types=[
            pltpu.VMEM((_CH,), jnp.int32),
            pltpu.VMEM((_R, _CH), jnp.float32),
            pltpu.VMEM((_R, _CH), jnp.float32),
            pltpu.VMEM((_R, _CH), jnp.float32),
            pltpu.VMEM((_R, _CH), jnp.float32),
            pltpu.SemaphoreType.DMA,
            pltpu.SemaphoreType.DMA,
            pltpu.SemaphoreType.DMA,
            pltpu.SemaphoreType.DMA,
        ],
    )
    def k(x_hbm, perm_hbm, out_hbm, perm_v, in0, in1, out0, out1, is0, is1, os0, os1):
        wid = lax.axis_index("s") * 2 + lax.axis_index("c")
        base = wid * _ROWS_PW
        pltpu.sync_copy(perm_hbm, perm_v)
        pltpu.async_copy(x_hbm.at[pl.ds(base, _R)], in0, is0)
        pltpu.async_copy(x_hbm.at[pl.ds(base + _R, _R)], in1, is1)

        def process(c, slot_in, slot_out, isem, osem, c0):
            pltpu.make_async_copy(
                x_hbm.at[pl.ds(0, _R)], slot_in, isem
            ).wait()

            @pl.when(c0 > 0)
            def _():
                pltpu.make_async_copy(
                    slot_out, out_hbm.at[pl.ds(0, _R)], osem
                ).wait()

            _gather_chunk(perm_v, slot_in, slot_out)
            pltpu.async_copy(
                slot_out, out_hbm.at[pl.ds(base + c * _R, _R)], osem
            )

            @pl.when(c0 < _NCHUNK // 2 - 1)
            def _():
                pltpu.async_copy(
                    x_hbm.at[pl.ds(base + (c + 2) * _R, _R)], slot_in, isem
                )

        def outer(c0, _):
            process(2 * c0, in0, out0, is0, os0, c0)
            process(2 * c0 + 1, in1, out1, is1, os1, c0)
            return _

        lax.fori_loop(0, _NCHUNK // 2, outer, None)
        pltpu.make_async_copy(out0, out_hbm.at[pl.ds(0, _R)], os0).wait()
        pltpu.make_async_copy(out1, out_hbm.at[pl.ds(0, _R)], os1).wait()

    return k(x, perm32)


def kernel(x, perm):
    return _sc_permute(x, perm.astype(jnp.int32))


# speedup vs baseline: 3.4979x; 3.4979x over previous
import functools

import jax
import jax.numpy as jnp
from jax import lax
from jax.experimental import pallas as pl
from jax.experimental.pallas import tpu as pltpu
from jax.experimental.pallas import tpu_sc as plsc


_BATCH = 8192
_CH = 4096
_LANES = 16
_NW = 32
_ROWS_PW = _BATCH // _NW
_R = 4
_NCHUNK = _ROWS_PW // _R
_NV = _CH // _LANES
_U = 4


def _gather_chunk(perm_v, inb, outb):
    @plsc.parallel_loop(0, _NV, unroll=_U)
    def v_body(v):
        off = v * _LANES
        idx = perm_v[pl.ds(off, _LANES)]
        for r in range(_R):
            rv = jnp.full((_LANES,), r, jnp.int32)
            outb[r, pl.ds(off, _LANES)] = plsc.load_gather(inb, [rv, idx])


def _sc_permute(x, perm32):
    mesh = plsc.VectorSubcoreMesh(core_axis_name="c", subcore_axis_name="s")

    @functools.partial(
        pl.kernel,
        mesh=mesh,
        compiler_params=pltpu.CompilerParams(needs_layout_passes=False),
        out_type=jax.ShapeDtypeStruct((_BATCH, _CH), jnp.float32),
        scratch_types=[
            pltpu.VMEM((_CH,), jnp.int32),
            pltpu.VMEM((_R, _CH), jnp.float32),
            pltpu.VMEM((_R, _CH), jnp.float32),
            pltpu.VMEM((_R, _CH), jnp.float32),
            pltpu.VMEM((_R, _CH), jnp.float32),
            pltpu.SemaphoreType.DMA,
            pltpu.SemaphoreType.DMA,
            pltpu.SemaphoreType.DMA,
            pltpu.SemaphoreType.DMA,
        ],
    )
    def k(x_hbm, perm_hbm, out_hbm, perm_v, in0, in1, out0, out1, is0, is1, os0, os1):
        wid = lax.axis_index("s") * 2 + lax.axis_index("c")
        base = wid * _ROWS_PW
        pltpu.sync_copy(perm_hbm, perm_v)
        pltpu.async_copy(x_hbm.at[pl.ds(base, _R)], in0, is0)
        pltpu.async_copy(x_hbm.at[pl.ds(base + _R, _R)], in1, is1)

        def process(c, slot_in, slot_out, isem, osem, c0):
            pltpu.make_async_copy(
                x_hbm.at[pl.ds(0, _R)], slot_in, isem
            ).wait()

            @pl.when(c0 > 0)
            def _():
                pltpu.make_async_copy(
                    slot_out, out_hbm.at[pl.ds(0, _R)], osem
                ).wait()

            _gather_chunk(perm_v, slot_in, slot_out)
            pltpu.async_copy(
                slot_out, out_hbm.at[pl.ds(base + c * _R, _R)], osem
            )

            @pl.when(c0 < _NCHUNK // 2 - 1)
            def _():
                pltpu.async_copy(
                    x_hbm.at[pl.ds(base + (c + 2) * _R, _R)], slot_in, isem
                )

        def outer(c0, _):
            process(2 * c0, in0, out0, is0, os0, c0)
            process(2 * c0 + 1, in1, out1, is1, os1, c0)
            return _

        lax.fori_loop(0, _NCHUNK // 2, outer, None)
        pltpu.make_async_copy(out0, out_hbm.at[pl.ds(0, _R)], os0).wait()
        pltpu.make_async_copy(out1, out_hbm.at[pl.ds(0, _R)], os1).wait()

    return k(x, perm32)


def kernel(x, perm):
    return _sc_permute(x, perm.astype(jnp.int32))


# speedup vs baseline: 3.6337x; 1.0388x over previous
import functools

import jax
import jax.numpy as jnp
from jax import lax
from jax.experimental import pallas as pl
from jax.experimental.pallas import tpu as pltpu
from jax.experimental.pallas import tpu_sc as plsc


_BATCH = 8192
_CH = 4096
_LANES = 16
_NW = 32
_ROWS_PW = _BATCH // _NW
_R = 4
_NCHUNK = _ROWS_PW // _R
_NV = _CH // _LANES
_U = 4
_DIN = 4
_DOUT = 2


def _gather_chunk(perm_v, inb, outb):
    @plsc.parallel_loop(0, _NV, unroll=_U)
    def v_body(v):
        off = v * _LANES
        idx = perm_v[pl.ds(off, _LANES)]
        for r in range(_R):
            rv = jnp.full((_LANES,), r, jnp.int32)
            outb[r, pl.ds(off, _LANES)] = plsc.load_gather(inb, [rv, idx])


def _sc_permute(x, perm32):
    mesh = plsc.VectorSubcoreMesh(core_axis_name="c", subcore_axis_name="s")

    scratch = (
        [pltpu.VMEM((_CH,), jnp.int32)]
        + [pltpu.VMEM((_R, _CH), jnp.float32)] * (_DIN + _DOUT)
        + [pltpu.SemaphoreType.DMA] * (_DIN + _DOUT)
    )

    @functools.partial(
        pl.kernel,
        mesh=mesh,
        compiler_params=pltpu.CompilerParams(needs_layout_passes=False),
        out_type=jax.ShapeDtypeStruct((_BATCH, _CH), jnp.float32),
        scratch_types=scratch,
    )
    def k(x_hbm, perm_hbm, out_hbm, perm_v, *bufs):
        ins = list(bufs[:_DIN])
        outs = list(bufs[_DIN:_DIN + _DOUT])
        sems = list(bufs[_DIN + _DOUT:])
        isems = sems[:_DIN]
        osems = sems[_DIN:]

        wid = lax.axis_index("s") * 2 + lax.axis_index("c")
        base = wid * _ROWS_PW
        pltpu.sync_copy(perm_hbm, perm_v)
        for d in range(_DIN):
            pltpu.async_copy(
                x_hbm.at[pl.ds(base + d * _R, _R)], ins[d], isems[d]
            )

        def outer(c4, _):
            for d in range(_DIN):
                c = c4 * _DIN + d
                o = d % _DOUT
                pltpu.make_async_copy(
                    x_hbm.at[pl.ds(0, _R)], ins[d], isems[d]
                ).wait()

                def wait_out():
                    pltpu.make_async_copy(
                        outs[o], out_hbm.at[pl.ds(0, _R)], osems[o]
                    ).wait()

                if d < _DOUT:
                    pl.when(c4 > 0)(wait_out)
                else:
                    wait_out()

                _gather_chunk(perm_v, ins[d], outs[o])
                pltpu.async_copy(
                    outs[o], out_hbm.at[pl.ds(base + c * _R, _R)], osems[o]
                )

                @pl.when(c4 < _NCHUNK // _DIN - 1)
                def _():
                    pltpu.async_copy(
                        x_hbm.at[pl.ds(base + (c + _DIN) * _R, _R)],
                        ins[d],
                        isems[d],
                    )
            return _

        lax.fori_loop(0, _NCHUNK // _DIN, outer, None)
        for o in range(_DOUT):
            pltpu.make_async_copy(
                outs[o], out_hbm.at[pl.ds(0, _R)], osems[o]
            ).wait()

    return k(x, perm32)


def kernel(x, perm):
    return _sc_permute(x, perm.astype(jnp.int32))
